# baseline (speedup 1.0000x reference)
import functools

import jax
import jax.numpy as jnp
import numpy as np
from jax import lax
from jax.experimental import pallas as pl
from jax.experimental.pallas import tpu as pltpu
from jax.experimental.pallas import tpu_sc as plsc

N = 128
H = 256
L = 6
S = 100
ED = 64
CUTOFF = 8.0

BI = 32
NBLK = N // BI
E_BLK = BI * N
NE = N * N
EAW = ED + BI + 2

_PREC = lax.Precision.HIGHEST


def _silu(x):
    return x * jax.nn.sigmoid(x)


def _dot(a, b):
    return jnp.dot(a, b, preferred_element_type=jnp.float32, precision=_PREC)


def _dot1(a, b):
    return jnp.dot(a, b, preferred_element_type=jnp.float32)


def _split(x):
    hi = x.astype(jnp.bfloat16)
    lo = (x - hi.astype(jnp.float32)).astype(jnp.bfloat16)
    return hi, lo


def _dot3(x, wh, wl):
    xh, xl = _split(x)
    return _dot1(xh, wh) + _dot1(xh, wl) + _dot1(xl, wh)


def _dotsel(sel, x):
    xh, xl = _split(x)
    return _dot1(sel, xh) + _dot1(sel, xl)


def _sc_gather(table, idx):
    info = plsc.get_sparse_core_info()
    nc = info.num_cores
    n_workers = 16
    rows_per = N // n_workers
    mesh = plsc.VectorSubcoreMesh(core_axis_name="c", subcore_axis_name="s")

    @functools.partial(
        pl.kernel,
        mesh=mesh,
        out_type=jax.ShapeDtypeStruct((N, H), jnp.float32),
        scratch_types=[
            pltpu.VMEM((rows_per,), jnp.int32),
            pltpu.VMEM((rows_per, H), jnp.float32),
            pltpu.SemaphoreType.DMA,
        ],
    )
    def gather_kernel(table_hbm, idx_hbm, out_hbm, idx_v, rows_v, sem):
        wid = lax.axis_index("s") * nc + lax.axis_index("c")

        @pl.when(wid < n_workers)
        def _():
            base = wid * rows_per
            pltpu.sync_copy(idx_hbm.at[pl.ds(base, rows_per)], idx_v)
            pltpu.async_copy(table_hbm.at[idx_v], rows_v, sem).wait()
            pltpu.sync_copy(rows_v, out_hbm.at[pl.ds(base, rows_per)])

    return gather_kernel(table, idx)


def _row_onehot(dtype=jnp.float32):
    er = lax.broadcasted_iota(jnp.int32, (E_BLK, BI), 0) // N
    return (er == lax.broadcasted_iota(jnp.int32, (E_BLK, BI), 1)).astype(dtype)


def _col_onehot(dtype=jnp.float32):
    ec = lax.broadcasted_iota(jnp.int32, (E_BLK, N), 0) % N
    return (ec == lax.broadcasted_iota(jnp.int32, (E_BLK, N), 1)).astype(dtype)


def _row_onehot_t(dtype=jnp.float32):
    er = lax.broadcasted_iota(jnp.int32, (BI, E_BLK), 1) // N
    return (er == lax.broadcasted_iota(jnp.int32, (BI, E_BLK), 0)).astype(dtype)


def _main_body(nf0, c0,
               t2, time_W2, tw1, tb1, tw2, tb2,
               edge_Wp, edge_ph, e1h, e1l, eb1, e2h, e2l, eb2,
               ewah, ewal, ewbh, ewbl, ewc, ew_bias, ew2h, ew2l, ew2_b,
               nwh, nwl, nw_b, nw2h, nw2l, nw2_b,
               cpw1, cpb1, cpw2, cpb2, tpw1, tpb1, tpw2, tpb2,
               cn_out, tl_out,
               nf, A, CB, eaHs, eaLs, RTs, CselS):
    l = pl.program_id(0)
    i = pl.program_id(1)
    r0 = i * BI
    e0 = i * E_BLK

    @pl.when((l == 0) & (i == 0))
    def _init():
        tp = t2[...] * time_W2[...]
        tf = jnp.concatenate([jnp.sin(tp), jnp.cos(tp)], axis=1)
        th = _silu(_dot(tf, tw1[...]) + tb1[...])
        te = _dot(th, tw2[...]) + tb2[...]
        nf[...] = nf0[...] + te
        RTs[...] = _row_onehot_t(jnp.bfloat16)
        CselS[...] = _col_onehot(jnp.bfloat16)

    @pl.when(l == 0)
    def _edge_cache():
        Rsel = _row_onehot()
        Csel = _col_onehot()
        cb = c0[pl.ds(r0, BI), :]
        cr = _dot(Rsel, cb)
        cc = _dot(Csel, c0[...])
        de = cr - cc
        d_col = jnp.sqrt(jnp.sum(de * de, axis=1, keepdims=True))

        er = r0 + lax.broadcasted_iota(jnp.int32, (E_BLK, 1), 0) // N
        ec = lax.broadcasted_iota(jnp.int32, (E_BLK, 1), 0) % N
        wm = jnp.where((d_col < CUTOFF) & (er != ec), 1.0, 0.0)

        xp = d_col * edge_Wp[...] + edge_ph[...]
        ea0 = jnp.sin(xp)
        hh = _silu(_dot3(ea0, e1h[...], e1l[...]) + eb1[...])
        ea = _dot3(hh, e2h[...], e2l[...]) + eb2[...]
        ones = jnp.ones((E_BLK, 1), jnp.float32)
        eaR = jnp.concatenate([ea, Rsel, ones, wm], axis=1)
        hi, lo = _split(eaR)
        eaHs[pl.ds(e0, E_BLK), :] = hi
        eaLs[pl.ds(e0, E_BLK), :] = lo

    @pl.when(i == 0)
    def _per_layer():
        nfv = nf[...]
        A[...] = _dot3(nfv, ewah[0], ewal[0])
        Bm = _dot3(nfv, ewbh[0], ewbl[0])
        CB[...] = _dotsel(CselS[...], Bm)

    eaH = eaHs[pl.ds(e0, E_BLK), :]
    eaL = eaLs[pl.ds(e0, E_BLK), :]
    zrow = jnp.zeros((1, H), jnp.float32)
    rhs = jnp.concatenate(
        [ewc[0], A[pl.ds(r0, BI), :], ew_bias[0], zrow], axis=0)
    rh, rl = _split(rhs)
    pre = _dot1(eaH, rh) + _dot1(eaH, rl) + _dot1(eaL, rh) + CB[...]
    em = _dot3(_silu(pre), ew2h[0], ew2l[0]) + ew2_b[0]

    wmc = eaH[:, EAW - 1:EAW].astype(jnp.float32)
    nmsg = _dotsel(RTs[...], em * wmc)

    nfb = nf[pl.ds(r0, BI), :]
    hn = _silu(_dot3(jnp.concatenate([nfb, nmsg], axis=1), nwh[0], nwl[0])
               + nw_b[0])
    nfn = _dot3(hn, nw2h[0], nw2l[0]) + nw2_b[0]
    nf[pl.ds(r0, BI), :] = nfn

    @pl.when(l == L - 1)
    def _heads():
        hc = _silu(_dot(nfn, cpw1[...]) + cpb1[...])
        cn_out[...] = _dot(hc, cpw2[...]) + cpb2[...]
        ht = _silu(_dot(nfn, tpw1[...]) + tpb1[...])
        tl_out[...] = _dot(ht, tpw2[...]) + tpb2[...]


def kernel(coords, atom_types, t, batch, time_W, edge_W, params):
    p = params
    f32 = jnp.float32
    bf16 = jnp.bfloat16
    coords = coords.astype(f32)

    def split_w(w):
        hi = w.astype(bf16)
        lo = (w - hi.astype(f32)).astype(bf16)
        return hi, lo

    nf0 = _sc_gather(p['atom_table'], atom_types.astype(jnp.int32))

    c_nat = jnp.pad(coords, ((0, 0), (0, 5)))
    t2 = t.astype(f32).reshape(1, 1)
    time_W2 = (time_W * (2.0 * np.pi)).reshape(1, H // 2)
    eW = (edge_W * (2.0 * np.pi)).reshape(1, ED // 2)
    edge_Wp = jnp.concatenate([eW, eW], axis=1)
    edge_ph = jnp.concatenate(
        [jnp.zeros((1, ED // 2), f32),
         jnp.full((1, ED // 2), 0.5 * np.pi, f32)], axis=1)

    e1h, e1l = split_w(p['edge_w1'])
    e2h, e2l = split_w(p['edge_w2'])

    ewah, ewal = split_w(p['ew'][:, :H, :])
    ewbh, ewbl = split_w(p['ew'][:, H:2 * H, :])
    ewc = p['ew'][:, 2 * H:, :]
    ew_bias = p['ew_b'].reshape(L, 1, H)
    ew2h, ew2l = split_w(p['ew2'])
    ew2_b = p['ew2_b'].reshape(L, 1, H)
    nwh, nwl = split_w(p['nw'])
    nw_b = p['nw_b'].reshape(L, 1, H)
    nw2h, nw2l = split_w(p['nw2'])
    nw2_b = p['nw2_b'].reshape(L, 1, H)

    cpw2 = jnp.pad(p['cp_w2'], ((0, 0), (0, 128 - 3)))
    cpb2 = jnp.pad(p['cp_b2'].reshape(1, 3), ((0, 0), (0, 128 - 3)))
    tpw2 = jnp.pad(p['tp_w2'], ((0, 0), (0, 128 - S)))
    tpb2 = jnp.pad(p['tp_b2'].reshape(1, S), ((0, 0), (0, 128 - S)))

    wspec = lambda: pl.BlockSpec((1, H, H), lambda l, i: (l, 0, 0))
    bspec = lambda: pl.BlockSpec((1, 1, H), lambda l, i: (l, 0, 0))
    cspec = lambda shape: pl.BlockSpec(shape, lambda l, i: tuple(0 for _ in shape))

    cn_full, tl_full = pl.pallas_call(
        _main_body,
        grid=(L, NBLK),
        in_specs=[
            cspec((N, H)),
            cspec((N, 8)),
            cspec((1, 1)), cspec((1, H // 2)),
            cspec((H, 4 * H)), cspec((1, 4 * H)),
            cspec((4 * H, H)), cspec((1, H)),
            cspec((1, ED)), cspec((1, ED)),
            cspec((ED, ED)), cspec((ED, ED)), cspec((1, ED)),
            cspec((ED, ED)), cspec((ED, ED)), cspec((1, ED)),
            wspec(), wspec(), wspec(), wspec(),
            pl.BlockSpec((1, ED, H), lambda l, i: (l, 0, 0)),
            bspec(),
            wspec(), wspec(), bspec(),
            pl.BlockSpec((1, 2 * H, H), lambda l, i: (l, 0, 0)),
            pl.BlockSpec((1, 2 * H, H), lambda l, i: (l, 0, 0)),
            bspec(),
            wspec(), wspec(), bspec(),
            cspec((H, H)), cspec((1, H)),
            cspec((H, 128)), cspec((1, 128)),
            cspec((H, H)), cspec((1, H)),
            cspec((H, 128)), cspec((1, 128)),
        ],
        out_specs=[
            pl.BlockSpec((BI, 128), lambda l, i: (i, 0)),
            pl.BlockSpec((BI, 128), lambda l, i: (i, 0)),
        ],
        out_shape=[
            jax.ShapeDtypeStruct((N, 128), f32),
            jax.ShapeDtypeStruct((N, 128), f32),
        ],
        scratch_shapes=[
            pltpu.VMEM((N, H), f32),
            pltpu.VMEM((N, H), f32),
            pltpu.VMEM((E_BLK, H), f32),
            pltpu.VMEM((NE, EAW), bf16),
            pltpu.VMEM((NE, EAW), bf16),
            pltpu.VMEM((BI, E_BLK), bf16),
            pltpu.VMEM((E_BLK, N), bf16),
        ],
        compiler_params=pltpu.CompilerParams(
            dimension_semantics=("arbitrary", "arbitrary")),
    )(nf0, c_nat,
      t2, time_W2,
      p['time_w1'], p['time_b1'].reshape(1, 4 * H),
      p['time_w2'], p['time_b2'].reshape(1, H),
      edge_Wp, edge_ph,
      e1h, e1l, p['edge_b1'].reshape(1, ED),
      e2h, e2l, p['edge_b2'].reshape(1, ED),
      ewah, ewal, ewbh, ewbl, ewc, ew_bias, ew2h, ew2l, ew2_b,
      nwh, nwl, nw_b, nw2h, nw2l, nw2_b,
      p['cp_w1'], p['cp_b1'].reshape(1, H), cpw2, cpb2,
      p['tp_w1'], p['tp_b1'].reshape(1, H), tpw2, tpb2)

    return cn_full[:, :3], tl_full[:, :S]


# speedup vs baseline: 1.0132x; 1.0132x over previous
import functools

import jax
import jax.numpy as jnp
import numpy as np
from jax import lax
from jax.experimental import pallas as pl
from jax.experimental.pallas import tpu as pltpu
from jax.experimental.pallas import tpu_sc as plsc

N = 128
H = 256
L = 6
S = 100
ED = 64
CUTOFF = 8.0

BI = 32
NBLK = N // BI
E_BLK = BI * N
NE = N * N
EAW = ED + BI + 2

_PREC = lax.Precision.HIGHEST


def _silu(x):
    return x * jax.nn.sigmoid(x)


def _dot(a, b):
    return jnp.dot(a, b, preferred_element_type=jnp.float32, precision=_PREC)


def _dot1(a, b):
    return jnp.dot(a, b, preferred_element_type=jnp.float32)


def _split(x):
    hi = x.astype(jnp.bfloat16)
    lo = (x - hi.astype(jnp.float32)).astype(jnp.bfloat16)
    return hi, lo


def _dot3(x, wh, wl):
    xh, xl = _split(x)
    return _dot1(xh, wh) + _dot1(xh, wl) + _dot1(xl, wh)


def _dotsel(sel, x):
    xh, xl = _split(x)
    return _dot1(sel, xh) + _dot1(sel, xl)


def _sc_gather(table, idx):
    info = plsc.get_sparse_core_info()
    nc = info.num_cores
    n_workers = 16
    rows_per = N // n_workers
    mesh = plsc.VectorSubcoreMesh(core_axis_name="c", subcore_axis_name="s")

    @functools.partial(
        pl.kernel,
        mesh=mesh,
        out_type=jax.ShapeDtypeStruct((N, H), jnp.float32),
        scratch_types=[
            pltpu.VMEM((rows_per,), jnp.int32),
            pltpu.VMEM((rows_per, H), jnp.float32),
            pltpu.SemaphoreType.DMA,
        ],
    )
    def gather_kernel(table_hbm, idx_hbm, out_hbm, idx_v, rows_v, sem):
        wid = lax.axis_index("s") * nc + lax.axis_index("c")

        @pl.when(wid < n_workers)
        def _():
            base = wid * rows_per
            pltpu.sync_copy(idx_hbm.at[pl.ds(base, rows_per)], idx_v)
            pltpu.async_copy(table_hbm.at[idx_v], rows_v, sem).wait()
            pltpu.sync_copy(rows_v, out_hbm.at[pl.ds(base, rows_per)])

    return gather_kernel(table, idx)


def _row_onehot(dtype=jnp.float32):
    er = lax.broadcasted_iota(jnp.int32, (E_BLK, BI), 0) // N
    return (er == lax.broadcasted_iota(jnp.int32, (E_BLK, BI), 1)).astype(dtype)


def _col_onehot(dtype=jnp.float32):
    ec = lax.broadcasted_iota(jnp.int32, (E_BLK, N), 0) % N
    return (ec == lax.broadcasted_iota(jnp.int32, (E_BLK, N), 1)).astype(dtype)


def _row_onehot_t(dtype=jnp.float32):
    er = lax.broadcasted_iota(jnp.int32, (BI, E_BLK), 1) // N
    return (er == lax.broadcasted_iota(jnp.int32, (BI, E_BLK), 0)).astype(dtype)


def _main_body(nf0, c0,
               t2, time_W2, tw1, tb1, tw2, tb2,
               edge_Wp, edge_ph, e1h, e1l, eb1, e2h, e2l, eb2,
               ewah, ewal, ewbh, ewbl, ewc, ew_bias, ew2h, ew2l, ew2_b,
               nwah, nwal, nwmh, nwml, nw_b, nw2h, nw2l, nw2_b,
               cpw1, cpb1, cpw2, cpb2, tpw1, tpb1, tpw2, tpb2,
               cn_out, tl_out,
               nf, A, CB, eaHs, eaLs, RTs, CselS):
    l = pl.program_id(0)
    i = pl.program_id(1)
    r0 = i * BI
    e0 = i * E_BLK

    @pl.when((l == 0) & (i == 0))
    def _init():
        tp = t2[...] * time_W2[...]
        tf = jnp.concatenate([jnp.sin(tp), jnp.cos(tp)], axis=1)
        th = _silu(_dot(tf, tw1[...]) + tb1[...])
        te = _dot(th, tw2[...]) + tb2[...]
        nf[...] = nf0[...] + te
        RTs[...] = _row_onehot_t(jnp.bfloat16)
        CselS[...] = _col_onehot(jnp.bfloat16)

    @pl.when(l == 0)
    def _edge_cache():
        Rsel = _row_onehot()
        Csel = _col_onehot()
        cb = c0[pl.ds(r0, BI), :]
        cr = _dot(Rsel, cb)
        cc = _dot(Csel, c0[...])
        de = cr - cc
        d_col = jnp.sqrt(jnp.sum(de * de, axis=1, keepdims=True))

        er = r0 + lax.broadcasted_iota(jnp.int32, (E_BLK, 1), 0) // N
        ec = lax.broadcasted_iota(jnp.int32, (E_BLK, 1), 0) % N
        wm = jnp.where((d_col < CUTOFF) & (er != ec), 1.0, 0.0)

        xp = d_col * edge_Wp[...] + edge_ph[...]
        ea0 = jnp.sin(xp)
        hh = _silu(_dot3(ea0, e1h[...], e1l[...]) + eb1[...])
        ea = _dot3(hh, e2h[...], e2l[...]) + eb2[...]
        ones = jnp.ones((E_BLK, 1), jnp.float32)
        eaR = jnp.concatenate([ea, Rsel, ones, wm], axis=1)
        hi, lo = _split(eaR)
        eaHs[pl.ds(e0, E_BLK), :] = hi
        eaLs[pl.ds(e0, E_BLK), :] = lo

    @pl.when(i == 0)
    def _per_layer():
        nfv = nf[...]
        A[...] = _dot3(nfv, ewah[0], ewal[0])
        Bm = _dot3(nfv, ewbh[0], ewbl[0])
        CB[...] = _dotsel(CselS[...], Bm)

    eaH = eaHs[pl.ds(e0, E_BLK), :]
    eaL = eaLs[pl.ds(e0, E_BLK), :]
    zrow = jnp.zeros((1, H), jnp.float32)
    rhs = jnp.concatenate(
        [ewc[0], A[pl.ds(r0, BI), :], ew_bias[0], zrow], axis=0)
    rh, rl = _split(rhs)
    pre = _dot1(eaH, rh) + _dot1(eaH, rl) + _dot1(eaL, rh) + CB[...]
    em = _dot3(_silu(pre), ew2h[0], ew2l[0]) + ew2_b[0]

    wmc = eaH[:, EAW - 1:EAW].astype(jnp.float32)
    nmsg = _dotsel(RTs[...], em * wmc)

    nfb = nf[pl.ds(r0, BI), :]
    hn = _silu(_dot3(nfb, nwah[0], nwal[0])
               + _dot3(nmsg, nwmh[0], nwml[0]) + nw_b[0])
    nfn = _dot3(hn, nw2h[0], nw2l[0]) + nw2_b[0]
    nf[pl.ds(r0, BI), :] = nfn

    @pl.when(l == L - 1)
    def _heads():
        hc = _silu(_dot(nfn, cpw1[...]) + cpb1[...])
        cn_out[...] = _dot(hc, cpw2[...]) + cpb2[...]
        ht = _silu(_dot(nfn, tpw1[...]) + tpb1[...])
        tl_out[...] = _dot(ht, tpw2[...]) + tpb2[...]


def kernel(coords, atom_types, t, batch, time_W, edge_W, params):
    p = params
    f32 = jnp.float32
    bf16 = jnp.bfloat16
    coords = coords.astype(f32)

    def split_w(w):
        hi = w.astype(bf16)
        lo = (w - hi.astype(f32)).astype(bf16)
        return hi, lo

    nf0 = _sc_gather(p['atom_table'], atom_types.astype(jnp.int32))

    c_nat = jnp.pad(coords, ((0, 0), (0, 5)))
    t2 = t.astype(f32).reshape(1, 1)
    time_W2 = (time_W * (2.0 * np.pi)).reshape(1, H // 2)
    eW = (edge_W * (2.0 * np.pi)).reshape(1, ED // 2)
    edge_Wp = jnp.concatenate([eW, eW], axis=1)
    edge_ph = jnp.concatenate(
        [jnp.zeros((1, ED // 2), f32),
         jnp.full((1, ED // 2), 0.5 * np.pi, f32)], axis=1)

    e1h, e1l = split_w(p['edge_w1'])
    e2h, e2l = split_w(p['edge_w2'])

    ewah, ewal = split_w(p['ew'][:, :H, :])
    ewbh, ewbl = split_w(p['ew'][:, H:2 * H, :])
    ewc = p['ew'][:, 2 * H:, :]
    ew_bias = p['ew_b'].reshape(L, 1, H)
    ew2h, ew2l = split_w(p['ew2'])
    ew2_b = p['ew2_b'].reshape(L, 1, H)
    nwah, nwal = split_w(p['nw'][:, :H, :])
    nwmh, nwml = split_w(p['nw'][:, H:, :])
    nw_b = p['nw_b'].reshape(L, 1, H)
    nw2h, nw2l = split_w(p['nw2'])
    nw2_b = p['nw2_b'].reshape(L, 1, H)

    cpw2 = jnp.pad(p['cp_w2'], ((0, 0), (0, 128 - 3)))
    cpb2 = jnp.pad(p['cp_b2'].reshape(1, 3), ((0, 0), (0, 128 - 3)))
    tpw2 = jnp.pad(p['tp_w2'], ((0, 0), (0, 128 - S)))
    tpb2 = jnp.pad(p['tp_b2'].reshape(1, S), ((0, 0), (0, 128 - S)))

    wspec = lambda: pl.BlockSpec((1, H, H), lambda l, i: (l, 0, 0))
    bspec = lambda: pl.BlockSpec((1, 1, H), lambda l, i: (l, 0, 0))
    cspec = lambda shape: pl.BlockSpec(shape, lambda l, i: tuple(0 for _ in shape))

    cn_full, tl_full = pl.pallas_call(
        _main_body,
        grid=(L, NBLK),
        in_specs=[
            cspec((N, H)),
            cspec((N, 8)),
            cspec((1, 1)), cspec((1, H // 2)),
            cspec((H, 4 * H)), cspec((1, 4 * H)),
            cspec((4 * H, H)), cspec((1, H)),
            cspec((1, ED)), cspec((1, ED)),
            cspec((ED, ED)), cspec((ED, ED)), cspec((1, ED)),
            cspec((ED, ED)), cspec((ED, ED)), cspec((1, ED)),
            wspec(), wspec(), wspec(), wspec(),
            pl.BlockSpec((1, ED, H), lambda l, i: (l, 0, 0)),
            bspec(),
            wspec(), wspec(), bspec(),
            wspec(), wspec(), wspec(), wspec(), bspec(),
            wspec(), wspec(), bspec(),
            cspec((H, H)), cspec((1, H)),
            cspec((H, 128)), cspec((1, 128)),
            cspec((H, H)), cspec((1, H)),
            cspec((H, 128)), cspec((1, 128)),
        ],
        out_specs=[
            pl.BlockSpec((BI, 128), lambda l, i: (i, 0)),
            pl.BlockSpec((BI, 128), lambda l, i: (i, 0)),
        ],
        out_shape=[
            jax.ShapeDtypeStruct((N, 128), f32),
            jax.ShapeDtypeStruct((N, 128), f32),
        ],
        scratch_shapes=[
            pltpu.VMEM((N, H), f32),
            pltpu.VMEM((N, H), f32),
            pltpu.VMEM((E_BLK, H), f32),
            pltpu.VMEM((NE, EAW), bf16),
            pltpu.VMEM((NE, EAW), bf16),
            pltpu.VMEM((BI, E_BLK), bf16),
            pltpu.VMEM((E_BLK, N), bf16),
        ],
        compiler_params=pltpu.CompilerParams(
            dimension_semantics=("arbitrary", "arbitrary")),
    )(nf0, c_nat,
      t2, time_W2,
      p['time_w1'], p['time_b1'].reshape(1, 4 * H),
      p['time_w2'], p['time_b2'].reshape(1, H),
      edge_Wp, edge_ph,
      e1h, e1l, p['edge_b1'].reshape(1, ED),
      e2h, e2l, p['edge_b2'].reshape(1, ED),
      ewah, ewal, ewbh, ewbl, ewc, ew_bias, ew2h, ew2l, ew2_b,
      nwah, nwal, nwmh, nwml, nw_b, nw2h, nw2l, nw2_b,
      p['cp_w1'], p['cp_b1'].reshape(1, H), cpw2, cpb2,
      p['tp_w1'], p['tp_b1'].reshape(1, H), tpw2, tpb2)

    return cn_full[:, :3], tl_full[:, :S]
